# baseline (device time: 74331 ns/iter reference)
import jax
import jax.numpy as jnp
from jax import lax
from jax.experimental import pallas as pl
from jax.experimental.pallas import tpu as pltpu

N_DEV = 4
N_GLOBAL = 8192
EPS = 1e-5
M = 6144
ROWS_C = 48
BLK = 4
MB = 512
N_BLOCKS = M // MB
LANES = 128


def _body(x_ref, gamma_ref, out_ref, stash_ref, obuf_ref, psum_ref, comm_ref,
          scale_ref, scol_ref, in_sems, out_sems, send_sems, recv_sems):
    me = lax.axis_index("i")
    nl = x_ref.shape[1]

    def in_copy(b):
        sl = pl.ds(MB * b, MB)
        return pltpu.make_async_copy(x_ref.at[sl], stash_ref.at[sl], in_sems.at[b])

    def out_copy(b):
        sl = pl.ds(MB * b, MB)
        return pltpu.make_async_copy(obuf_ref.at[b % 2], out_ref.at[sl], out_sems.at[b])

    for b in range(N_BLOCKS):
        in_copy(b).start()

    for b in range(N_BLOCKS):
        in_copy(b).wait()
        xx = stash_ref[pl.ds(MB * b, MB)]
        x2 = xx * xx
        f = x2[:, 0:LANES]
        for k in range(1, nl // LANES):
            f = f + x2[:, LANES * k:LANES * (k + 1)]
        f3 = f.reshape(BLK, 128, 128)
        psum_ref[pl.ds(BLK * b, BLK), :] = jnp.sum(f3, axis=2)

    barrier = pltpu.get_barrier_semaphore()
    for k in range(1, N_DEV):
        peer = (me + k) % N_DEV
        pl.semaphore_signal(
            barrier, inc=1,
            device_id=(peer,), device_id_type=pl.DeviceIdType.MESH,
        )
    pl.semaphore_wait(barrier, N_DEV - 1)

    comm_ref[me] = psum_ref[...]

    sends = []
    for k in range(1, N_DEV):
        peer = (me + k) % N_DEV
        rdma = pltpu.make_async_remote_copy(
            src_ref=comm_ref.at[me],
            dst_ref=comm_ref.at[me],
            send_sem=send_sems.at[k - 1],
            recv_sem=recv_sems.at[me],
            device_id=(peer,),
            device_id_type=pl.DeviceIdType.MESH,
        )
        rdma.start()
        sends.append(rdma)

    for k in range(1, N_DEV):
        peer = (me + k) % N_DEV
        recv = pltpu.make_async_remote_copy(
            src_ref=comm_ref.at[peer],
            dst_ref=comm_ref.at[peer],
            send_sem=send_sems.at[k - 1],
            recv_sem=recv_sems.at[peer],
            device_id=(peer,),
            device_id_type=pl.DeviceIdType.MESH,
        )
        recv.wait_recv()
    for s in sends:
        s.wait_send()

    total = comm_ref[0] + comm_ref[1] + comm_ref[2] + comm_ref[3]
    scale_ref[...] = lax.rsqrt(total * (1.0 / N_GLOBAL) + EPS)

    eye = (lax.broadcasted_iota(jnp.int32, (128, 128), 0)
           == lax.broadcasted_iota(jnp.int32, (128, 128), 1)
           ).astype(jnp.float32)
    for j in range(ROWS_C):
        row = scale_ref[pl.ds(j, 1), :]
        scol_ref[pl.ds(128 * j, 128), :] = jnp.sum(
            eye * row, axis=1, keepdims=True)

    g = gamma_ref[...]
    for b in range(N_BLOCKS):
        if b >= 2:
            out_copy(b - 2).wait()
        s = scol_ref[pl.ds(MB * b, MB), :]
        obuf_ref[b % 2] = stash_ref[pl.ds(MB * b, MB)] * g * s
        out_copy(b).start()

    for b in range(N_BLOCKS - 2, N_BLOCKS):
        out_copy(b).wait()


def kernel(x, gamma):
    m, n_local = x.shape
    gamma2 = gamma.reshape(1, n_local)

    return pl.pallas_call(
        _body,
        in_specs=[
            pl.BlockSpec(memory_space=pltpu.MemorySpace.HBM),
            pl.BlockSpec(memory_space=pltpu.VMEM),
        ],
        out_specs=pl.BlockSpec(memory_space=pltpu.MemorySpace.HBM),
        out_shape=jax.ShapeDtypeStruct((m, n_local), jnp.float32),
        scratch_shapes=[
            pltpu.VMEM((m, n_local), jnp.float32),
            pltpu.VMEM((2, MB, n_local), jnp.float32),
            pltpu.VMEM((ROWS_C, 128), jnp.float32),
            pltpu.VMEM((N_DEV, ROWS_C, 128), jnp.float32),
            pltpu.VMEM((ROWS_C, 128), jnp.float32),
            pltpu.VMEM((m, 1), jnp.float32),
            pltpu.SemaphoreType.DMA((N_BLOCKS,)),
            pltpu.SemaphoreType.DMA((N_BLOCKS,)),
            pltpu.SemaphoreType.DMA((N_DEV - 1,)),
            pltpu.SemaphoreType.DMA((N_DEV,)),
        ],
        compiler_params=pltpu.CompilerParams(
            collective_id=0,
            vmem_limit_bytes=100 * 1024 * 1024,
        ),
    )(x, gamma2)


# device time: 57658 ns/iter; 1.2892x vs baseline; 1.2892x over previous
import jax
import jax.numpy as jnp
from jax import lax
from jax.experimental import pallas as pl
from jax.experimental.pallas import tpu as pltpu

N_DEV = 4
N_GLOBAL = 8192
EPS = 1e-5
M = 6144
ROWS_C = 48
BLK = 4
MB = 512
N_BLOCKS = M // MB
LANES = 128


def _partial_body(x_ref, psum_ref, stash_ref, in_sems):
    def in_copy(b):
        sl = pl.ds(MB * b, MB)
        return pltpu.make_async_copy(x_ref.at[sl], stash_ref.at[sl], in_sems.at[b])

    for b in range(N_BLOCKS):
        in_copy(b).start()

    for b in range(N_BLOCKS):
        in_copy(b).wait()
        xx = stash_ref[pl.ds(MB * b, MB)]
        x2 = xx * xx
        f = x2[:, 0:LANES]
        for k in range(1, x2.shape[1] // LANES):
            f = f + x2[:, LANES * k:LANES * (k + 1)]
        f3 = f.reshape(BLK, 128, 128)
        psum_ref[pl.ds(BLK * b, BLK), :] = jnp.sum(f3, axis=2)


def _allreduce_body(p_ref, scale_ref, comm_ref, send_sems, recv_sems):
    me = lax.axis_index("i")

    barrier = pltpu.get_barrier_semaphore()
    for k in range(1, N_DEV):
        peer = (me + k) % N_DEV
        pl.semaphore_signal(
            barrier, inc=1,
            device_id=(peer,), device_id_type=pl.DeviceIdType.MESH,
        )
    pl.semaphore_wait(barrier, N_DEV - 1)

    comm_ref[me] = p_ref[...]

    sends = []
    for k in range(1, N_DEV):
        peer = (me + k) % N_DEV
        rdma = pltpu.make_async_remote_copy(
            src_ref=comm_ref.at[me],
            dst_ref=comm_ref.at[me],
            send_sem=send_sems.at[k - 1],
            recv_sem=recv_sems.at[me],
            device_id=(peer,),
            device_id_type=pl.DeviceIdType.MESH,
        )
        rdma.start()
        sends.append(rdma)

    for k in range(1, N_DEV):
        peer = (me + k) % N_DEV
        recv = pltpu.make_async_remote_copy(
            src_ref=comm_ref.at[peer],
            dst_ref=comm_ref.at[peer],
            send_sem=send_sems.at[k - 1],
            recv_sem=recv_sems.at[peer],
            device_id=(peer,),
            device_id_type=pl.DeviceIdType.MESH,
        )
        recv.wait_recv()
    for s in sends:
        s.wait_send()

    total = comm_ref[0] + comm_ref[1] + comm_ref[2] + comm_ref[3]
    scale_ref[...] = lax.rsqrt(total * (1.0 / N_GLOBAL) + EPS)


def _norm_body(x_ref, gamma_ref, scale_ref, out_ref, stash_ref, obuf_ref,
               scol_ref, in_sems, out_sems):
    def in_copy(b):
        sl = pl.ds(MB * b, MB)
        return pltpu.make_async_copy(x_ref.at[sl], stash_ref.at[sl], in_sems.at[b])

    def out_copy(b):
        sl = pl.ds(MB * b, MB)
        return pltpu.make_async_copy(obuf_ref.at[b % 2], out_ref.at[sl], out_sems.at[b])

    for b in range(N_BLOCKS):
        in_copy(b).start()

    eye = (lax.broadcasted_iota(jnp.int32, (128, 128), 0)
           == lax.broadcasted_iota(jnp.int32, (128, 128), 1)
           ).astype(jnp.float32)
    for j in range(ROWS_C):
        row = scale_ref[pl.ds(j, 1), :]
        scol_ref[pl.ds(128 * j, 128), :] = jnp.sum(
            eye * row, axis=1, keepdims=True)

    g = gamma_ref[...]
    for b in range(N_BLOCKS):
        if b >= 2:
            out_copy(b - 2).wait()
        in_copy(b).wait()
        s = scol_ref[pl.ds(MB * b, MB), :]
        obuf_ref[b % 2] = stash_ref[pl.ds(MB * b, MB)] * g * s
        out_copy(b).start()

    for b in range(N_BLOCKS - 2, N_BLOCKS):
        out_copy(b).wait()


def kernel(x, gamma):
    m, n_local = x.shape
    gamma2 = gamma.reshape(1, n_local)

    psum = pl.pallas_call(
        _partial_body,
        in_specs=[pl.BlockSpec(memory_space=pltpu.MemorySpace.HBM)],
        out_specs=pl.BlockSpec(memory_space=pltpu.VMEM),
        out_shape=jax.ShapeDtypeStruct((ROWS_C, 128), jnp.float32),
        scratch_shapes=[
            pltpu.VMEM((m, n_local), jnp.float32),
            pltpu.SemaphoreType.DMA((N_BLOCKS,)),
        ],
        compiler_params=pltpu.CompilerParams(
            vmem_limit_bytes=100 * 1024 * 1024,
        ),
    )(x)

    scale_c = pl.pallas_call(
        _allreduce_body,
        in_specs=[pl.BlockSpec(memory_space=pltpu.VMEM)],
        out_specs=pl.BlockSpec(memory_space=pltpu.VMEM),
        out_shape=jax.ShapeDtypeStruct((ROWS_C, 128), jnp.float32),
        scratch_shapes=[
            pltpu.VMEM((N_DEV, ROWS_C, 128), jnp.float32),
            pltpu.SemaphoreType.DMA((N_DEV - 1,)),
            pltpu.SemaphoreType.DMA((N_DEV,)),
        ],
        compiler_params=pltpu.CompilerParams(collective_id=0),
    )(psum)

    return pl.pallas_call(
        _norm_body,
        in_specs=[
            pl.BlockSpec(memory_space=pltpu.MemorySpace.HBM),
            pl.BlockSpec(memory_space=pltpu.VMEM),
            pl.BlockSpec(memory_space=pltpu.VMEM),
        ],
        out_specs=pl.BlockSpec(memory_space=pltpu.MemorySpace.HBM),
        out_shape=jax.ShapeDtypeStruct((m, n_local), jnp.float32),
        scratch_shapes=[
            pltpu.VMEM((m, n_local), jnp.float32),
            pltpu.VMEM((2, MB, n_local), jnp.float32),
            pltpu.VMEM((m, 1), jnp.float32),
            pltpu.SemaphoreType.DMA((N_BLOCKS,)),
            pltpu.SemaphoreType.DMA((N_BLOCKS,)),
        ],
        compiler_params=pltpu.CompilerParams(
            vmem_limit_bytes=100 * 1024 * 1024,
        ),
    )(x, gamma2, scale_c)
